# initial kernel scaffold (unmeasured)
import jax
import jax.numpy as jnp
from jax import lax
from jax.experimental import pallas as pl
from jax.experimental.pallas import tpu as pltpu

N_DEV = 4
N_HOPS = 2 * (N_DEV - 1)
SUB = 512


def kernel(x, w_mat):
    m = x.shape[0]
    n = w_mat.shape[1]
    chunk = m // N_DEV

    partial = jnp.dot(x, w_mat, preferred_element_type=jnp.float32)

    def body(p_ref, out_ref, comm_ref, acc_ref, tmp_ref, send_sems, recv_sems,
             local_sems):
        my = lax.axis_index("i")
        right = lax.rem(my + 1, N_DEV)
        left = lax.rem(my + N_DEV - 1, N_DEV)

        barrier = pltpu.get_barrier_semaphore()
        for nbr in (left, right):
            pl.semaphore_signal(barrier, inc=1, device_id=(nbr,),
                                device_id_type=pl.DeviceIdType.MESH)
        pl.semaphore_wait(barrier, 2)

        cp = pltpu.make_async_copy(p_ref, out_ref, local_sems.at[0])
        cp.start()
        cp.wait()

        for s in range(N_DEV - 1):
            send_idx = lax.rem(my - s + N_DEV, N_DEV)
            recv_idx = lax.rem(my - s - 1 + N_DEV, N_DEV)
            rdma = pltpu.make_async_remote_copy(
                src_ref=out_ref.at[pl.ds(send_idx * chunk, chunk), :],
                dst_ref=comm_ref.at[s],
                send_sem=send_sems.at[s],
                recv_sem=recv_sems.at[s],
                device_id=(right,),
                device_id_type=pl.DeviceIdType.MESH,
            )
            rdma.start()
            rdma.wait()

            for t in range(chunk // SUB):
                row0 = recv_idx * chunk + t * SUB
                c_acc = pltpu.make_async_copy(
                    out_ref.at[pl.ds(row0, SUB), :], acc_ref, local_sems.at[0])
                c_tmp = pltpu.make_async_copy(
                    comm_ref.at[s, pl.ds(t * SUB, SUB), :], tmp_ref,
                    local_sems.at[1])
                c_acc.start()
                c_tmp.start()
                c_acc.wait()
                c_tmp.wait()
                acc_ref[...] = acc_ref[...] + tmp_ref[...]
                c_out = pltpu.make_async_copy(
                    acc_ref, out_ref.at[pl.ds(row0, SUB), :], local_sems.at[0])
                c_out.start()
                c_out.wait()

        for g in range(N_DEV - 1):
            h = (N_DEV - 1) + g
            send_idx = lax.rem(my + 1 - g + N_DEV, N_DEV)
            rdma = pltpu.make_async_remote_copy(
                src_ref=out_ref.at[pl.ds(send_idx * chunk, chunk), :],
                dst_ref=out_ref.at[pl.ds(send_idx * chunk, chunk), :],
                send_sem=send_sems.at[h],
                recv_sem=recv_sems.at[h],
                device_id=(right,),
                device_id_type=pl.DeviceIdType.MESH,
            )
            rdma.start()
            rdma.wait()

    return pl.pallas_call(
        body,
        out_shape=jax.ShapeDtypeStruct((m, n), jnp.float32),
        in_specs=[pl.BlockSpec(memory_space=pl.ANY)],
        out_specs=pl.BlockSpec(memory_space=pl.ANY),
        scratch_shapes=[
            pltpu.MemorySpace.HBM((N_DEV - 1, chunk, n), jnp.float32),
            pltpu.MemorySpace.VMEM((SUB, n), jnp.float32),
            pltpu.MemorySpace.VMEM((SUB, n), jnp.float32),
            pltpu.SemaphoreType.DMA((N_HOPS,)),
            pltpu.SemaphoreType.DMA((N_HOPS,)),
            pltpu.SemaphoreType.DMA((2,)),
        ],
        compiler_params=pltpu.CompilerParams(collective_id=0),
    )(partial)


# baseline (device time: 6556657 ns/iter reference)
import jax
import jax.numpy as jnp
from jax import lax
from jax.experimental import pallas as pl
from jax.experimental.pallas import tpu as pltpu

N_DEV = 4
N_HOPS = 2 * (N_DEV - 1)
SUB = 512


def kernel(x, w_mat):
    m = x.shape[0]
    n = w_mat.shape[1]
    chunk = m // N_DEV

    partial = jnp.dot(x, w_mat, preferred_element_type=jnp.float32)
    comm = jnp.zeros((N_DEV - 1, chunk, n), jnp.float32)

    def body(p_ref, comm_ref, out_ref, acc_ref, tmp_ref, send_sems, recv_sems,
             local_sems):
        my = lax.axis_index("i")
        right = lax.rem(my + 1, N_DEV)
        left = lax.rem(my + N_DEV - 1, N_DEV)

        barrier = pltpu.get_barrier_semaphore()
        for nbr in (left, right):
            pl.semaphore_signal(barrier, inc=1, device_id=(nbr,),
                                device_id_type=pl.DeviceIdType.MESH)
        pl.semaphore_wait(barrier, 2)

        cp = pltpu.make_async_copy(p_ref, out_ref, local_sems.at[0])
        cp.start()
        cp.wait()

        for s in range(N_DEV - 1):
            send_idx = lax.rem(my - s + N_DEV, N_DEV)
            recv_idx = lax.rem(my - s - 1 + N_DEV, N_DEV)
            rdma = pltpu.make_async_remote_copy(
                src_ref=out_ref.at[pl.ds(send_idx * chunk, chunk), :],
                dst_ref=comm_ref.at[s],
                send_sem=send_sems.at[s],
                recv_sem=recv_sems.at[s],
                device_id=(right,),
                device_id_type=pl.DeviceIdType.MESH,
            )
            rdma.start()
            rdma.wait()

            for t in range(chunk // SUB):
                row0 = recv_idx * chunk + t * SUB
                c_acc = pltpu.make_async_copy(
                    out_ref.at[pl.ds(row0, SUB), :], acc_ref, local_sems.at[0])
                c_tmp = pltpu.make_async_copy(
                    comm_ref.at[s, pl.ds(t * SUB, SUB), :], tmp_ref,
                    local_sems.at[1])
                c_acc.start()
                c_tmp.start()
                c_acc.wait()
                c_tmp.wait()
                acc_ref[...] = acc_ref[...] + tmp_ref[...]
                c_out = pltpu.make_async_copy(
                    acc_ref, out_ref.at[pl.ds(row0, SUB), :], local_sems.at[0])
                c_out.start()
                c_out.wait()

        for g in range(N_DEV - 1):
            h = (N_DEV - 1) + g
            send_idx = lax.rem(my + 1 - g + N_DEV, N_DEV)
            rdma = pltpu.make_async_remote_copy(
                src_ref=out_ref.at[pl.ds(send_idx * chunk, chunk), :],
                dst_ref=out_ref.at[pl.ds(send_idx * chunk, chunk), :],
                send_sem=send_sems.at[h],
                recv_sem=recv_sems.at[h],
                device_id=(right,),
                device_id_type=pl.DeviceIdType.MESH,
            )
            rdma.start()
            rdma.wait()

    return pl.pallas_call(
        body,
        out_shape=jax.ShapeDtypeStruct((m, n), jnp.float32),
        in_specs=[pl.BlockSpec(memory_space=pl.ANY),
                  pl.BlockSpec(memory_space=pl.ANY)],
        out_specs=pl.BlockSpec(memory_space=pl.ANY),
        scratch_shapes=[
            pltpu.MemorySpace.VMEM((SUB, n), jnp.float32),
            pltpu.MemorySpace.VMEM((SUB, n), jnp.float32),
            pltpu.SemaphoreType.DMA((N_HOPS,)),
            pltpu.SemaphoreType.DMA((N_HOPS,)),
            pltpu.SemaphoreType.DMA((2,)),
        ],
        compiler_params=pltpu.CompilerParams(collective_id=0),
    )(partial, comm)


# device time: 1411626 ns/iter; 4.6448x vs baseline; 4.6448x over previous
import jax
import jax.numpy as jnp
from jax import lax
from jax.experimental import pallas as pl
from jax.experimental.pallas import tpu as pltpu

N_DEV = 4
N_HOPS = 2 * (N_DEV - 1)
SUB = 512


def kernel(x, w_mat):
    m = x.shape[0]
    n = w_mat.shape[1]
    chunk = m // N_DEV
    half = n // 2

    partial = jnp.dot(x, w_mat, preferred_element_type=jnp.float32)
    comm = jnp.zeros((2, N_DEV - 1, chunk, half), jnp.float32)

    def body(p_ref, comm_ref, out_ref, acc_ref, tmp_ref, send_sems, recv_sems,
             local_sems):
        del p_ref
        my = lax.axis_index("i")
        right = lax.rem(my + 1, N_DEV)
        left = lax.rem(my + N_DEV - 1, N_DEV)

        barrier = pltpu.get_barrier_semaphore()
        for nbr in (left, right):
            pl.semaphore_signal(barrier, inc=1, device_id=(nbr,),
                                device_id_type=pl.DeviceIdType.MESH)
        pl.semaphore_wait(barrier, 2)

        def accum(ring, s, ridx, col0):
            for t in range(chunk // SUB):
                row0 = ridx * chunk + t * SUB
                c_acc = pltpu.make_async_copy(
                    out_ref.at[pl.ds(row0, SUB), pl.ds(col0, half)],
                    acc_ref, local_sems.at[0])
                c_tmp = pltpu.make_async_copy(
                    comm_ref.at[ring, s, pl.ds(t * SUB, SUB), :],
                    tmp_ref, local_sems.at[1])
                c_acc.start()
                c_tmp.start()
                c_acc.wait()
                c_tmp.wait()
                acc_ref[...] = acc_ref[...] + tmp_ref[...]
                c_out = pltpu.make_async_copy(
                    acc_ref,
                    out_ref.at[pl.ds(row0, SUB), pl.ds(col0, half)],
                    local_sems.at[0])
                c_out.start()
                c_out.wait()

        for s in range(N_DEV - 1):
            sR = lax.rem(my - s + N_DEV, N_DEV)
            rR = lax.rem(my - s - 1 + N_DEV, N_DEV)
            sL = lax.rem(my + s, N_DEV)
            rL = lax.rem(my + s + 1, N_DEV)
            rdma_r = pltpu.make_async_remote_copy(
                src_ref=out_ref.at[pl.ds(sR * chunk, chunk), pl.ds(0, half)],
                dst_ref=comm_ref.at[0, s],
                send_sem=send_sems.at[0, s], recv_sem=recv_sems.at[0, s],
                device_id=(right,), device_id_type=pl.DeviceIdType.MESH)
            rdma_l = pltpu.make_async_remote_copy(
                src_ref=out_ref.at[pl.ds(sL * chunk, chunk), pl.ds(half, half)],
                dst_ref=comm_ref.at[1, s],
                send_sem=send_sems.at[1, s], recv_sem=recv_sems.at[1, s],
                device_id=(left,), device_id_type=pl.DeviceIdType.MESH)
            rdma_r.start()
            rdma_l.start()
            rdma_r.wait()
            accum(0, s, rR, 0)
            rdma_l.wait()
            accum(1, s, rL, half)

        for g in range(N_DEV - 1):
            h = (N_DEV - 1) + g
            sR = lax.rem(my + 1 - g + N_DEV, N_DEV)
            sL = lax.rem(my - 1 + g + N_DEV, N_DEV)
            rdma_r = pltpu.make_async_remote_copy(
                src_ref=out_ref.at[pl.ds(sR * chunk, chunk), pl.ds(0, half)],
                dst_ref=out_ref.at[pl.ds(sR * chunk, chunk), pl.ds(0, half)],
                send_sem=send_sems.at[0, h], recv_sem=recv_sems.at[0, h],
                device_id=(right,), device_id_type=pl.DeviceIdType.MESH)
            rdma_l = pltpu.make_async_remote_copy(
                src_ref=out_ref.at[pl.ds(sL * chunk, chunk), pl.ds(half, half)],
                dst_ref=out_ref.at[pl.ds(sL * chunk, chunk), pl.ds(half, half)],
                send_sem=send_sems.at[1, h], recv_sem=recv_sems.at[1, h],
                device_id=(left,), device_id_type=pl.DeviceIdType.MESH)
            rdma_r.start()
            rdma_l.start()
            rdma_r.wait()
            rdma_l.wait()

    return pl.pallas_call(
        body,
        out_shape=jax.ShapeDtypeStruct((m, n), jnp.float32),
        in_specs=[pl.BlockSpec(memory_space=pl.ANY),
                  pl.BlockSpec(memory_space=pl.ANY)],
        out_specs=pl.BlockSpec(memory_space=pl.ANY),
        scratch_shapes=[
            pltpu.MemorySpace.VMEM((SUB, half), jnp.float32),
            pltpu.MemorySpace.VMEM((SUB, half), jnp.float32),
            pltpu.SemaphoreType.DMA((2, N_HOPS)),
            pltpu.SemaphoreType.DMA((2, N_HOPS)),
            pltpu.SemaphoreType.DMA((2,)),
        ],
        input_output_aliases={0: 0},
        compiler_params=pltpu.CompilerParams(collective_id=0),
    )(partial, comm)


# device time: 1410281 ns/iter; 4.6492x vs baseline; 1.0010x over previous
import jax
import jax.numpy as jnp
from jax import lax
from jax.experimental import pallas as pl
from jax.experimental.pallas import tpu as pltpu

N_DEV = 4
N_HOPS = 2 * (N_DEV - 1)
SUB = 512


def kernel(x, w_mat):
    m = x.shape[0]
    n = w_mat.shape[1]
    chunk = m // N_DEV
    half = n // 2

    partial = jnp.dot(x, w_mat, preferred_element_type=jnp.float32)
    comm = jnp.zeros((2, N_DEV - 1, chunk, half), jnp.float32)

    def body(p_ref, comm_ref, out_ref, acc_ref, tmp_ref, send_sems, recv_sems,
             local_sems):
        my = lax.axis_index("i")
        right = lax.rem(my + 1, N_DEV)
        left = lax.rem(my + N_DEV - 1, N_DEV)

        barrier = pltpu.get_barrier_semaphore()
        for nbr in (left, right):
            pl.semaphore_signal(barrier, inc=1, device_id=(nbr,),
                                device_id_type=pl.DeviceIdType.MESH)
        pl.semaphore_wait(barrier, 2)

        def accum(ring, s, ridx, col0):
            for t in range(chunk // SUB):
                row0 = ridx * chunk + t * SUB
                c_acc = pltpu.make_async_copy(
                    p_ref.at[pl.ds(row0, SUB), pl.ds(col0, half)],
                    acc_ref, local_sems.at[0])
                c_tmp = pltpu.make_async_copy(
                    comm_ref.at[ring, s, pl.ds(t * SUB, SUB), :],
                    tmp_ref, local_sems.at[1])
                c_acc.start()
                c_tmp.start()
                c_acc.wait()
                c_tmp.wait()
                acc_ref[...] = acc_ref[...] + tmp_ref[...]
                c_out = pltpu.make_async_copy(
                    acc_ref,
                    out_ref.at[pl.ds(row0, SUB), pl.ds(col0, half)],
                    local_sems.at[0])
                c_out.start()
                c_out.wait()

        for s in range(N_DEV - 1):
            sR = lax.rem(my - s + N_DEV, N_DEV)
            rR = lax.rem(my - s - 1 + N_DEV, N_DEV)
            sL = lax.rem(my + s, N_DEV)
            rL = lax.rem(my + s + 1, N_DEV)
            src = p_ref if s == 0 else out_ref
            rdma_r = pltpu.make_async_remote_copy(
                src_ref=src.at[pl.ds(sR * chunk, chunk), pl.ds(0, half)],
                dst_ref=comm_ref.at[0, s],
                send_sem=send_sems.at[0, s], recv_sem=recv_sems.at[0, s],
                device_id=(right,), device_id_type=pl.DeviceIdType.MESH)
            rdma_l = pltpu.make_async_remote_copy(
                src_ref=src.at[pl.ds(sL * chunk, chunk), pl.ds(half, half)],
                dst_ref=comm_ref.at[1, s],
                send_sem=send_sems.at[1, s], recv_sem=recv_sems.at[1, s],
                device_id=(left,), device_id_type=pl.DeviceIdType.MESH)
            rdma_r.start()
            rdma_l.start()
            rdma_r.wait()
            accum(0, s, rR, 0)
            rdma_l.wait()
            accum(1, s, rL, half)

        for g in range(N_DEV - 1):
            h = (N_DEV - 1) + g
            sR = lax.rem(my + 1 - g + N_DEV, N_DEV)
            sL = lax.rem(my - 1 + g + N_DEV, N_DEV)
            rdma_r = pltpu.make_async_remote_copy(
                src_ref=out_ref.at[pl.ds(sR * chunk, chunk), pl.ds(0, half)],
                dst_ref=out_ref.at[pl.ds(sR * chunk, chunk), pl.ds(0, half)],
                send_sem=send_sems.at[0, h], recv_sem=recv_sems.at[0, h],
                device_id=(right,), device_id_type=pl.DeviceIdType.MESH)
            rdma_l = pltpu.make_async_remote_copy(
                src_ref=out_ref.at[pl.ds(sL * chunk, chunk), pl.ds(half, half)],
                dst_ref=out_ref.at[pl.ds(sL * chunk, chunk), pl.ds(half, half)],
                send_sem=send_sems.at[1, h], recv_sem=recv_sems.at[1, h],
                device_id=(left,), device_id_type=pl.DeviceIdType.MESH)
            rdma_r.start()
            rdma_l.start()
            rdma_r.wait()
            rdma_l.wait()

    return pl.pallas_call(
        body,
        out_shape=jax.ShapeDtypeStruct((m, n), jnp.float32),
        in_specs=[pl.BlockSpec(memory_space=pl.ANY),
                  pl.BlockSpec(memory_space=pl.ANY)],
        out_specs=pl.BlockSpec(memory_space=pl.ANY),
        scratch_shapes=[
            pltpu.MemorySpace.VMEM((SUB, half), jnp.float32),
            pltpu.MemorySpace.VMEM((SUB, half), jnp.float32),
            pltpu.SemaphoreType.DMA((2, N_HOPS)),
            pltpu.SemaphoreType.DMA((2, N_HOPS)),
            pltpu.SemaphoreType.DMA((2,)),
        ],
        compiler_params=pltpu.CompilerParams(collective_id=0),
    )(partial, comm)


# device time: 1251980 ns/iter; 5.2370x vs baseline; 1.1264x over previous
import jax
import jax.numpy as jnp
from jax import lax
from jax.experimental import pallas as pl
from jax.experimental.pallas import tpu as pltpu

N_DEV = 4
N_HOPS = 2 * (N_DEV - 1)
BLK = 512


def kernel(x, w_mat):
    m = x.shape[0]
    n = w_mat.shape[1]
    chunk = m // N_DEV
    half = n // 2
    nblk = chunk // BLK

    partial = jnp.dot(x, w_mat, preferred_element_type=jnp.float32)

    def body(p_ref, out_ref, comm_ref, acc_ref, tmp_ref, send_sems, recv_sems,
             local_sems):
        my = lax.axis_index("i")
        right = lax.rem(my + 1, N_DEV)
        left = lax.rem(my + N_DEV - 1, N_DEV)

        def mod4(v):
            return lax.rem(v + 2 * N_DEV, N_DEV)

        def send_idx(d, h):
            if h < 3:
                return mod4(my - h) if d == 0 else mod4(my + h)
            g = h - 3
            return mod4(my + 1 - g) if d == 0 else mod4(my - 1 + g)

        def accum_idx(d, s):
            return mod4(my - s - 1) if d == 0 else mod4(my + s + 1)

        def recv_idx(d, g):
            return mod4(my - g) if d == 0 else mod4(my + g)

        def mk_rdma(d, h, t, chunk_idx):
            col0 = 0 if d == 0 else half
            rows = pl.ds(chunk_idx * chunk + t * BLK, BLK)
            src = (p_ref if h == 0 else out_ref).at[rows, pl.ds(col0, half)]
            if h < 3:
                dst = comm_ref.at[d, h, pl.ds(t * BLK, BLK), :]
            else:
                dst = out_ref.at[rows, pl.ds(col0, half)]
            return pltpu.make_async_remote_copy(
                src_ref=src, dst_ref=dst,
                send_sem=send_sems.at[d, h, t],
                recv_sem=recv_sems.at[d, h, t],
                device_id=(right if d == 0 else left,),
                device_id_type=pl.DeviceIdType.MESH)

        def accum(d, s, t):
            col0 = 0 if d == 0 else half
            row0 = accum_idx(d, s) * chunk + t * BLK
            c_p = pltpu.make_async_copy(
                p_ref.at[pl.ds(row0, BLK), pl.ds(col0, half)],
                acc_ref, local_sems.at[0])
            c_c = pltpu.make_async_copy(
                comm_ref.at[d, s, pl.ds(t * BLK, BLK), :],
                tmp_ref, local_sems.at[1])
            c_p.start()
            c_c.start()
            c_p.wait()
            c_c.wait()
            acc_ref[...] = acc_ref[...] + tmp_ref[...]
            c_o = pltpu.make_async_copy(
                acc_ref, out_ref.at[pl.ds(row0, BLK), pl.ds(col0, half)],
                local_sems.at[0])
            c_o.start()
            c_o.wait()

        barrier = pltpu.get_barrier_semaphore()
        for nbr in (left, right):
            pl.semaphore_signal(barrier, inc=1, device_id=(nbr,),
                                device_id_type=pl.DeviceIdType.MESH)
        pl.semaphore_wait(barrier, 2)

        sends = []

        def start_send(d, h, t):
            r = mk_rdma(d, h, t, send_idx(d, h))
            r.start()
            sends.append(r)

        for t in range(nblk):
            for d in (0, 1):
                start_send(d, 0, t)

        for s in range(N_DEV - 1):
            for t in range(nblk):
                for d in (0, 1):
                    mk_rdma(d, s, t, accum_idx(d, s)).wait_recv()
                    accum(d, s, t)
                    start_send(d, s + 1, t)

        for g in range(N_DEV - 1):
            h = 3 + g
            for t in range(nblk):
                for d in (0, 1):
                    mk_rdma(d, h, t, recv_idx(d, g)).wait_recv()
                    if h < N_HOPS - 1:
                        start_send(d, h + 1, t)

        for r in sends:
            r.wait_send()

    out, _ = pl.pallas_call(
        body,
        out_shape=(
            jax.ShapeDtypeStruct((m, n), jnp.float32),
            jax.ShapeDtypeStruct((2, N_DEV - 1, chunk, half), jnp.float32),
        ),
        in_specs=[pl.BlockSpec(memory_space=pl.ANY)],
        out_specs=[pl.BlockSpec(memory_space=pl.ANY),
                   pl.BlockSpec(memory_space=pl.ANY)],
        scratch_shapes=[
            pltpu.MemorySpace.VMEM((BLK, half), jnp.float32),
            pltpu.MemorySpace.VMEM((BLK, half), jnp.float32),
            pltpu.SemaphoreType.DMA((2, N_HOPS, 2)),
            pltpu.SemaphoreType.DMA((2, N_HOPS, 2)),
            pltpu.SemaphoreType.DMA((2,)),
        ],
        compiler_params=pltpu.CompilerParams(collective_id=0),
    )(partial)
    return out


# device time: 1208852 ns/iter; 5.4239x vs baseline; 1.0357x over previous
import jax
import jax.numpy as jnp
from jax import lax
from jax.experimental import pallas as pl
from jax.experimental.pallas import tpu as pltpu

N_DEV = 4
N_HOPS = 2 * (N_DEV - 1)
BLK = 512


def kernel(x, w_mat):
    m = x.shape[0]
    ksh = x.shape[1]
    n = w_mat.shape[1]
    chunk = m // N_DEV
    half = n // 2
    nblk = chunk // BLK

    def body(x_ref, w_ref, out_ref, comm_ref, p_ref, xv, wv, ov, acc_ref,
             tmp_ref, send_sems, recv_sems, local_sems):
        my = lax.axis_index("i")
        right = lax.rem(my + 1, N_DEV)
        left = lax.rem(my + N_DEV - 1, N_DEV)

        def mod4(v):
            return lax.rem(v + 2 * N_DEV, N_DEV)

        def send_idx(d, h):
            if h < 3:
                return mod4(my - h) if d == 0 else mod4(my + h)
            g = h - 3
            return mod4(my + 1 - g) if d == 0 else mod4(my - 1 + g)

        def accum_idx(d, s):
            return mod4(my - s - 1) if d == 0 else mod4(my + s + 1)

        def recv_idx(d, g):
            return mod4(my - g) if d == 0 else mod4(my + g)

        def mk_rdma(d, h, t, chunk_idx):
            col0 = 0 if d == 0 else half
            rows = pl.ds(chunk_idx * chunk + t * BLK, BLK)
            src = (p_ref if h == 0 else out_ref).at[rows, pl.ds(col0, half)]
            if h < 3:
                dst = comm_ref.at[d, h, pl.ds(t * BLK, BLK), :]
            else:
                dst = out_ref.at[rows, pl.ds(col0, half)]
            return pltpu.make_async_remote_copy(
                src_ref=src, dst_ref=dst,
                send_sem=send_sems.at[d, h, t],
                recv_sem=recv_sems.at[d, h, t],
                device_id=(right if d == 0 else left,),
                device_id_type=pl.DeviceIdType.MESH)

        sends = []

        def start_send(d, h, t):
            r = mk_rdma(d, h, t, send_idx(d, h))
            r.start()
            sends.append(r)

        _loaded = {"x": None, "w": None}

        def load_w(j):
            if _loaded["w"] == j:
                return
            _loaded["w"] = j
            cp = pltpu.make_async_copy(
                w_ref.at[:, pl.ds(j * half, half)], wv, local_sems.at[1])
            cp.start()
            cp.wait()

        def tile(ckey, c, j, b=None):
            if _loaded["x"] != ckey:
                _loaded["x"] = ckey
                cp = pltpu.make_async_copy(
                    x_ref.at[pl.ds(c * chunk, chunk), :], xv,
                    local_sems.at[0])
                cp.start()
                cp.wait()
            load_w(j)
            rows = slice(None) if b is None else slice(b * BLK, (b + 1) * BLK)
            ov[rows, :] = jnp.dot(xv[rows, :], wv[...],
                                  preferred_element_type=jnp.float32)
            nrows = chunk if b is None else BLK
            r0 = c * chunk + (0 if b is None else b * BLK)
            st = pltpu.make_async_copy(
                ov.at[rows, :],
                p_ref.at[pl.ds(r0, nrows), pl.ds(j * half, half)],
                local_sems.at[2])
            st.start()
            st.wait()

        def accum(d, s, t):
            col0 = 0 if d == 0 else half
            row0 = accum_idx(d, s) * chunk + t * BLK
            c_p = pltpu.make_async_copy(
                p_ref.at[pl.ds(row0, BLK), pl.ds(col0, half)],
                acc_ref, local_sems.at[0])
            c_c = pltpu.make_async_copy(
                comm_ref.at[d, s, pl.ds(t * BLK, BLK), :],
                tmp_ref, local_sems.at[1])
            c_p.start()
            c_c.start()
            c_p.wait()
            c_c.wait()
            acc_ref[...] = acc_ref[...] + tmp_ref[...]
            c_o = pltpu.make_async_copy(
                acc_ref, out_ref.at[pl.ds(row0, BLK), pl.ds(col0, half)],
                local_sems.at[2])
            c_o.start()
            c_o.wait()

        barrier = pltpu.get_barrier_semaphore()
        for nbr in (left, right):
            pl.semaphore_signal(barrier, inc=1, device_id=(nbr,),
                                device_id_type=pl.DeviceIdType.MESH)
        pl.semaphore_wait(barrier, 2)

        for j in (0, 1):
            for b in range(nblk):
                tile("my", my, j, b)
                start_send(j, 0, b)

        tile("my-1", mod4(my - 1), 0)
        tile("my+1", mod4(my + 1), 1)

        for s in range(N_DEV - 1):
            for t in range(nblk):
                for d in (0, 1):
                    mk_rdma(d, s, t, accum_idx(d, s)).wait_recv()
                    accum(d, s, t)
                    start_send(d, s + 1, t)
            if s == 0:
                tile("my+2", mod4(my + 2), 0)
                tile("my+2", mod4(my + 2), 1)
            elif s == 1:
                tile("my+1", mod4(my + 1), 0)
                tile("my-1", mod4(my - 1), 1)

        for g in range(N_DEV - 1):
            h = 3 + g
            for t in range(nblk):
                for d in (0, 1):
                    mk_rdma(d, h, t, recv_idx(d, g)).wait_recv()
                    if h < N_HOPS - 1:
                        start_send(d, h + 1, t)

        for r in sends:
            r.wait_send()

    out, _, _ = pl.pallas_call(
        body,
        out_shape=(
            jax.ShapeDtypeStruct((m, n), jnp.float32),
            jax.ShapeDtypeStruct((2, N_DEV - 1, chunk, half), jnp.float32),
            jax.ShapeDtypeStruct((m, n), jnp.float32),
        ),
        in_specs=[pl.BlockSpec(memory_space=pl.ANY),
                  pl.BlockSpec(memory_space=pl.ANY)],
        out_specs=[pl.BlockSpec(memory_space=pl.ANY),
                   pl.BlockSpec(memory_space=pl.ANY),
                   pl.BlockSpec(memory_space=pl.ANY)],
        scratch_shapes=[
            pltpu.MemorySpace.VMEM((chunk, ksh), jnp.float32),
            pltpu.MemorySpace.VMEM((ksh, half), jnp.float32),
            pltpu.MemorySpace.VMEM((chunk, half), jnp.float32),
            pltpu.MemorySpace.VMEM((BLK, half), jnp.float32),
            pltpu.MemorySpace.VMEM((BLK, half), jnp.float32),
            pltpu.SemaphoreType.DMA((2, N_HOPS, 2)),
            pltpu.SemaphoreType.DMA((2, N_HOPS, 2)),
            pltpu.SemaphoreType.DMA((3,)),
        ],
        compiler_params=pltpu.CompilerParams(
            collective_id=0, vmem_limit_bytes=60 * 1024 * 1024),
    )(x, w_mat)
    return out


# device time: 1196601 ns/iter; 5.4794x vs baseline; 1.0102x over previous
import jax
import jax.numpy as jnp
from jax import lax
from jax.experimental import pallas as pl
from jax.experimental.pallas import tpu as pltpu

N_DEV = 4
N_HOPS = 2 * (N_DEV - 1)
BLK = 512


def kernel(x, w_mat):
    m = x.shape[0]
    ksh = x.shape[1]
    n = w_mat.shape[1]
    chunk = m // N_DEV
    half = n // 2
    nblk = chunk // BLK

    def body(x_ref, w_ref, out_ref, comm_ref, p_ref, xv, wv, ov, acc_ref,
             tmp_ref, send_sems, recv_sems, local_sems):
        my = lax.axis_index("i")
        right = lax.rem(my + 1, N_DEV)
        left = lax.rem(my + N_DEV - 1, N_DEV)

        def mod4(v):
            return lax.rem(v + 2 * N_DEV, N_DEV)

        def send_idx(d, h):
            if h < 3:
                return mod4(my - h) if d == 0 else mod4(my + h)
            g = h - 3
            return mod4(my + 1 - g) if d == 0 else mod4(my - 1 + g)

        def accum_idx(d, s):
            return mod4(my - s - 1) if d == 0 else mod4(my + s + 1)

        def recv_idx(d, g):
            return mod4(my - g) if d == 0 else mod4(my + g)

        def mk_rdma(d, h, t, chunk_idx):
            col0 = 0 if d == 0 else half
            rows = pl.ds(chunk_idx * chunk + t * BLK, BLK)
            src = (p_ref if h == 0 else out_ref).at[rows, pl.ds(col0, half)]
            if h < 3:
                dst = comm_ref.at[d, h, pl.ds(t * BLK, BLK), :]
            else:
                dst = out_ref.at[rows, pl.ds(col0, half)]
            return pltpu.make_async_remote_copy(
                src_ref=src, dst_ref=dst,
                send_sem=send_sems.at[d, h, t],
                recv_sem=recv_sems.at[d, h, t],
                device_id=(right if d == 0 else left,),
                device_id_type=pl.DeviceIdType.MESH)

        sends = []

        def start_send(d, h, t):
            r = mk_rdma(d, h, t, send_idx(d, h))
            r.start()
            sends.append(r)

        _loaded = {"x": None}

        def tile(ckey, c, j, b):
            if _loaded["x"] != ckey:
                _loaded["x"] = ckey
                cp = pltpu.make_async_copy(
                    x_ref.at[pl.ds(c * chunk, chunk), :], xv,
                    local_sems.at[0])
                cp.start()
                cp.wait()
            rows = slice(b * BLK, (b + 1) * BLK)
            ov[...] = jnp.dot(xv[rows, :], wv[:, j * half:(j + 1) * half],
                              preferred_element_type=jnp.float32)
            st = pltpu.make_async_copy(
                ov,
                p_ref.at[pl.ds(c * chunk + b * BLK, BLK),
                         pl.ds(j * half, half)],
                local_sems.at[2])
            st.start()
            st.wait()

        def accum(d, s, t):
            col0 = 0 if d == 0 else half
            row0 = accum_idx(d, s) * chunk + t * BLK
            c_p = pltpu.make_async_copy(
                p_ref.at[pl.ds(row0, BLK), pl.ds(col0, half)],
                acc_ref, local_sems.at[0])
            c_c = pltpu.make_async_copy(
                comm_ref.at[d, s, pl.ds(t * BLK, BLK), :],
                tmp_ref, local_sems.at[1])
            c_p.start()
            c_c.start()
            c_p.wait()
            c_c.wait()
            acc_ref[...] = acc_ref[...] + tmp_ref[...]
            c_o = pltpu.make_async_copy(
                acc_ref, out_ref.at[pl.ds(row0, BLK), pl.ds(col0, half)],
                local_sems.at[2])
            c_o.start()
            c_o.wait()

        w_cp = pltpu.make_async_copy(w_ref, wv, local_sems.at[1])
        w_cp.start()
        x_cp = pltpu.make_async_copy(
            x_ref.at[pl.ds(my * chunk, chunk), :], xv, local_sems.at[0])
        x_cp.start()
        _loaded["x"] = "my"

        barrier = pltpu.get_barrier_semaphore()
        for nbr in (left, right):
            pl.semaphore_signal(barrier, inc=1, device_id=(nbr,),
                                device_id_type=pl.DeviceIdType.MESH)
        pl.semaphore_wait(barrier, 2)
        x_cp.wait()
        w_cp.wait()

        for b in range(nblk):
            for j in (0, 1):
                tile("my", my, j, b)
                start_send(j, 0, b)

        for b in range(nblk):
            tile("my-1", mod4(my - 1), 0, b)
        for b in range(nblk):
            tile("my+1", mod4(my + 1), 1, b)

        for s in range(N_DEV - 1):
            for t in range(nblk):
                for d in (0, 1):
                    mk_rdma(d, s, t, accum_idx(d, s)).wait_recv()
                    accum(d, s, t)
                    start_send(d, s + 1, t)
            if s == 0:
                for j in (0, 1):
                    for b in range(nblk):
                        tile("my+2", mod4(my + 2), j, b)
            elif s == 1:
                for b in range(nblk):
                    tile("my+1", mod4(my + 1), 0, b)
                for b in range(nblk):
                    tile("my-1", mod4(my - 1), 1, b)

        for g in range(N_DEV - 1):
            h = 3 + g
            for t in range(nblk):
                for d in (0, 1):
                    mk_rdma(d, h, t, recv_idx(d, g)).wait_recv()
                    if h < N_HOPS - 1:
                        start_send(d, h + 1, t)

        for r in sends:
            r.wait_send()

    out, _, _ = pl.pallas_call(
        body,
        out_shape=(
            jax.ShapeDtypeStruct((m, n), jnp.float32),
            jax.ShapeDtypeStruct((2, N_DEV - 1, chunk, half), jnp.float32),
            jax.ShapeDtypeStruct((m, n), jnp.float32),
        ),
        in_specs=[pl.BlockSpec(memory_space=pl.ANY),
                  pl.BlockSpec(memory_space=pl.ANY)],
        out_specs=[pl.BlockSpec(memory_space=pl.ANY),
                   pl.BlockSpec(memory_space=pl.ANY),
                   pl.BlockSpec(memory_space=pl.ANY)],
        scratch_shapes=[
            pltpu.MemorySpace.VMEM((chunk, ksh), jnp.float32),
            pltpu.MemorySpace.VMEM((ksh, n), jnp.float32),
            pltpu.MemorySpace.VMEM((BLK, half), jnp.float32),
            pltpu.MemorySpace.VMEM((BLK, half), jnp.float32),
            pltpu.MemorySpace.VMEM((BLK, half), jnp.float32),
            pltpu.SemaphoreType.DMA((2, N_HOPS, 2)),
            pltpu.SemaphoreType.DMA((2, N_HOPS, 2)),
            pltpu.SemaphoreType.DMA((3,)),
        ],
        compiler_params=pltpu.CompilerParams(
            collective_id=0, vmem_limit_bytes=60 * 1024 * 1024),
    )(x, w_mat)
    return out
